# baseline (speedup 1.0000x reference)
import jax
import jax.numpy as jnp
from jax import lax
from jax.experimental import pallas as pl
from jax.experimental.pallas import tpu as pltpu
from jax.experimental.pallas import tpu_sc as plsc

_B, _C, _H, _W = 128, 3, 224, 224
_PX = _H * _W
_NCH = 14
_P = _PX // _NCH
_NBUF = 4

_info = plsc.get_sparse_core_info()
_NC, _NS = _info.num_cores, _info.num_subcores
_NW = _NC * _NS
_IPW = _B // _NW
_TOT = _IPW * _NCH
_GROUPS = _TOT // _NBUF


def _sc_body(x_hbm, w_hbm, o_hbm, wv, ibuf, obuf, isem, osem):
    wid = lax.axis_index("s") * _NC + lax.axis_index("c")
    base_img = wid * _IPW
    pltpu.sync_copy(w_hbm, wv)

    def chunk_coords(c):
        img = base_img + c // _NCH
        off = (c % _NCH) * _P
        return img, off

    for b in range(_NBUF):
        img, off = chunk_coords(b)
        pltpu.make_async_copy(
            x_hbm.at[img, :, pl.ds(off, _P)], ibuf.at[b], isem.at[b]).start()

    def group(g, carry):
        for b in range(_NBUF):
            c = g * _NBUF + b
            img, off = chunk_coords(c)
            pltpu.make_async_copy(
                x_hbm.at[img, :, pl.ds(off, _P)], ibuf.at[b], isem.at[b]).wait()

            @pl.when(c >= _NBUF)
            def _(b=b, c=c):
                pimg, poff = chunk_coords(c - _NBUF)
                pltpu.make_async_copy(
                    obuf.at[b], o_hbm.at[pimg, :, pl.ds(poff, _P)],
                    osem.at[b]).wait()

            msel = wv[img] > 0.5

            def px(j, pcarry, b=b, msel=msel):
                s0 = j * 16
                r = ibuf[b, 0, pl.ds(s0, 16)]
                g_ = ibuf[b, 1, pl.ds(s0, 16)]
                b_ = ibuf[b, 2, pl.ds(s0, 16)]
                lum = (r * (299.0 / 1000.0) + g_ * (587.0 / 1000.0)
                       + b_ * (114.0 / 1000.0))
                obuf[b, 0, pl.ds(s0, 16)] = jnp.where(msel, lum, r)
                obuf[b, 1, pl.ds(s0, 16)] = jnp.where(msel, lum, g_)
                obuf[b, 2, pl.ds(s0, 16)] = jnp.where(msel, lum, b_)
                return pcarry

            lax.fori_loop(0, _P // 16, px, 0)

            pltpu.make_async_copy(
                obuf.at[b], o_hbm.at[img, :, pl.ds(off, _P)], osem.at[b]).start()

            @pl.when(c + _NBUF < _TOT)
            def _(b=b, c=c):
                nimg, noff = chunk_coords(c + _NBUF)
                pltpu.make_async_copy(
                    x_hbm.at[nimg, :, pl.ds(noff, _P)], ibuf.at[b],
                    isem.at[b]).start()

        return carry

    lax.fori_loop(0, _GROUPS, group, 0)

    for b in range(_NBUF):
        img, off = chunk_coords(_TOT - _NBUF + b)
        pltpu.make_async_copy(
            obuf.at[b], o_hbm.at[img, :, pl.ds(off, _P)], osem.at[b]).wait()


@jax.jit
def _sc_gray(xr, w):
    mesh = plsc.VectorSubcoreMesh(core_axis_name="c", subcore_axis_name="s")
    return pl.kernel(
        _sc_body,
        out_type=jax.ShapeDtypeStruct((_B, _C, _PX), jnp.float32),
        mesh=mesh,
        scratch_types=[
            pltpu.VMEM((_B, 16), jnp.float32),
            pltpu.VMEM((_NBUF, _C, _P), jnp.float32),
            pltpu.VMEM((_NBUF, _C, _P), jnp.float32),
            pltpu.SemaphoreType.DMA((_NBUF,)),
            pltpu.SemaphoreType.DMA((_NBUF,)),
        ],
    )(xr, w)


def kernel(x, inds):
    xr = x.reshape(_B, _C, _PX)
    w = jnp.broadcast_to(inds.astype(jnp.float32)[:, None], (_B, 16))
    out = _sc_gray(xr, w)
    return out.reshape(_B, _C, _H, _W)


# speedup vs baseline: 1.0361x; 1.0361x over previous
import jax
import jax.numpy as jnp
from jax import lax
from jax.experimental import pallas as pl
from jax.experimental.pallas import tpu as pltpu
from jax.experimental.pallas import tpu_sc as plsc

_B, _C, _H, _W = 128, 3, 224, 224
_R = 16
_NCH = _H // _R
_NBUF = 4
_WV = _W // 16

_info = plsc.get_sparse_core_info()
_NC, _NS = _info.num_cores, _info.num_subcores
_NW = _NC * _NS
_IPW = _B // _NW
_TOT = _IPW * _NCH
_GROUPS = _TOT // _NBUF


def _sc_body(x_hbm, w_hbm, o_hbm, wv, ibuf, obuf, isem, osem):
    wid = lax.axis_index("s") * _NC + lax.axis_index("c")
    base_img = wid * _IPW
    pltpu.sync_copy(w_hbm, wv)

    def chunk_coords(c):
        img = base_img + c // _NCH
        row = (c % _NCH) * _R
        return img, row

    for b in range(_NBUF):
        img, row = chunk_coords(b)
        pltpu.make_async_copy(
            x_hbm.at[img, :, pl.ds(row, _R), :], ibuf.at[b], isem.at[b]).start()

    def group(g, carry):
        for b in range(_NBUF):
            c = g * _NBUF + b
            img, row = chunk_coords(c)
            pltpu.make_async_copy(
                x_hbm.at[img, :, pl.ds(row, _R), :], ibuf.at[b],
                isem.at[b]).wait()

            @pl.when(c >= _NBUF)
            def _(b=b, c=c):
                pimg, prow = chunk_coords(c - _NBUF)
                pltpu.make_async_copy(
                    obuf.at[b], o_hbm.at[pimg, :, pl.ds(prow, _R), :],
                    osem.at[b]).wait()

            msel = wv[img] > 0.5

            def px(j, pcarry, b=b, msel=msel):
                rr = j // _WV
                w0 = (j % _WV) * 16
                r = ibuf[b, 0, rr, pl.ds(w0, 16)]
                g_ = ibuf[b, 1, rr, pl.ds(w0, 16)]
                b_ = ibuf[b, 2, rr, pl.ds(w0, 16)]
                lum = (r * (299.0 / 1000.0) + g_ * (587.0 / 1000.0)
                       + b_ * (114.0 / 1000.0))
                obuf[b, 0, rr, pl.ds(w0, 16)] = jnp.where(msel, lum, r)
                obuf[b, 1, rr, pl.ds(w0, 16)] = jnp.where(msel, lum, g_)
                obuf[b, 2, rr, pl.ds(w0, 16)] = jnp.where(msel, lum, b_)
                return pcarry

            lax.fori_loop(0, _R * _WV, px, 0)

            pltpu.make_async_copy(
                obuf.at[b], o_hbm.at[img, :, pl.ds(row, _R), :],
                osem.at[b]).start()

            @pl.when(c + _NBUF < _TOT)
            def _(b=b, c=c):
                nimg, nrow = chunk_coords(c + _NBUF)
                pltpu.make_async_copy(
                    x_hbm.at[nimg, :, pl.ds(nrow, _R), :], ibuf.at[b],
                    isem.at[b]).start()

        return carry

    lax.fori_loop(0, _GROUPS, group, 0)

    for b in range(_NBUF):
        img, row = chunk_coords(_TOT - _NBUF + b)
        pltpu.make_async_copy(
            obuf.at[b], o_hbm.at[img, :, pl.ds(row, _R), :], osem.at[b]).wait()


@jax.jit
def _sc_gray(x, w):
    mesh = plsc.VectorSubcoreMesh(core_axis_name="c", subcore_axis_name="s")
    return pl.kernel(
        _sc_body,
        out_type=jax.ShapeDtypeStruct((_B, _C, _H, _W), jnp.float32),
        mesh=mesh,
        scratch_types=[
            pltpu.VMEM((_B, 16), jnp.float32),
            pltpu.VMEM((_NBUF, _C, _R, _W), jnp.float32),
            pltpu.VMEM((_NBUF, _C, _R, _W), jnp.float32),
            pltpu.SemaphoreType.DMA((_NBUF,)),
            pltpu.SemaphoreType.DMA((_NBUF,)),
        ],
        compiler_params=pltpu.CompilerParams(use_tc_tiling_on_sc=True),
    )(x, w)


def kernel(x, inds):
    w = jnp.broadcast_to(inds.astype(jnp.float32)[:, None], (_B, 16))
    return _sc_gray(x, w)


# speedup vs baseline: 2.6453x; 2.5531x over previous
import jax
import jax.numpy as jnp
from jax import lax
from jax.experimental import pallas as pl
from jax.experimental.pallas import tpu as pltpu
from jax.experimental.pallas import tpu_sc as plsc

_B, _C, _H, _W = 128, 3, 224, 224
_PX = _H * _W
_Q = 56
_NBUF = 2

_info = plsc.get_sparse_core_info()
_NC, _NS = _info.num_cores, _info.num_subcores
_NW = _NC * _NS
_PPW = _PX // _NW
_TOT = _PPW // _Q
_GROUPS = _TOT // _NBUF


def _sc_body(x_hbm, w_hbm, o_hbm, wv, ibuf, obuf, isem, osem):
    wid = lax.axis_index("s") * _NC + lax.axis_index("c")
    base_px = wid * _PPW
    pltpu.sync_copy(w_hbm, wv)

    for b in range(_NBUF):
        p0 = base_px + b * _Q
        pltpu.make_async_copy(
            x_hbm.at[:, pl.ds(p0, _Q), :], ibuf.at[b], isem.at[b]).start()

    def group(g, carry):
        for b in range(_NBUF):
            c = g * _NBUF + b
            p0 = base_px + c * _Q
            pltpu.make_async_copy(
                x_hbm.at[:, pl.ds(p0, _Q), :], ibuf.at[b], isem.at[b]).wait()

            @pl.when(c >= _NBUF)
            def _(b=b, c=c):
                pp0 = base_px + (c - _NBUF) * _Q
                pltpu.make_async_copy(
                    obuf.at[b], o_hbm.at[:, pl.ds(pp0, _Q), :],
                    osem.at[b]).wait()

            def px(j, pcarry, b=b):
                p = j // 8
                l0 = lax.rem(j, 8) * 16
                msel = wv[pl.ds(l0, 16)] > 0.5
                r = ibuf[b, 0, p, pl.ds(l0, 16)]
                g_ = ibuf[b, 1, p, pl.ds(l0, 16)]
                b_ = ibuf[b, 2, p, pl.ds(l0, 16)]
                lum = (r * (299.0 / 1000.0) + g_ * (587.0 / 1000.0)
                       + b_ * (114.0 / 1000.0))
                obuf[b, 0, p, pl.ds(l0, 16)] = jnp.where(msel, lum, r)
                obuf[b, 1, p, pl.ds(l0, 16)] = jnp.where(msel, lum, g_)
                obuf[b, 2, p, pl.ds(l0, 16)] = jnp.where(msel, lum, b_)
                return pcarry

            lax.fori_loop(0, _Q * 8, px, 0)

            pltpu.make_async_copy(
                obuf.at[b], o_hbm.at[:, pl.ds(p0, _Q), :], osem.at[b]).start()

            @pl.when(c + _NBUF < _TOT)
            def _(b=b, c=c):
                np0 = base_px + (c + _NBUF) * _Q
                pltpu.make_async_copy(
                    x_hbm.at[:, pl.ds(np0, _Q), :], ibuf.at[b],
                    isem.at[b]).start()

        return carry

    lax.fori_loop(0, _GROUPS, group, 0)

    for b in range(_NBUF):
        p0 = base_px + (_TOT - _NBUF + b) * _Q
        pltpu.make_async_copy(
            obuf.at[b], o_hbm.at[:, pl.ds(p0, _Q), :], osem.at[b]).wait()


@jax.jit
def _sc_gray(xt, w):
    mesh = plsc.VectorSubcoreMesh(core_axis_name="c", subcore_axis_name="s")
    return pl.kernel(
        _sc_body,
        out_type=jax.ShapeDtypeStruct((_C, _PX, _B), jnp.float32),
        mesh=mesh,
        scratch_types=[
            pltpu.VMEM((_B,), jnp.float32),
            pltpu.VMEM((_NBUF, _C, _Q, _B), jnp.float32),
            pltpu.VMEM((_NBUF, _C, _Q, _B), jnp.float32),
            pltpu.SemaphoreType.DMA((_NBUF,)),
            pltpu.SemaphoreType.DMA((_NBUF,)),
        ],
    )(xt, w)


def kernel(x, inds):
    xt = jnp.transpose(x, (1, 2, 3, 0)).reshape(_C, _PX, _B)
    out_t = _sc_gray(xt, inds.astype(jnp.float32))
    return jnp.transpose(out_t.reshape(_C, _H, _W, _B), (3, 0, 1, 2))


# speedup vs baseline: 4.0335x; 1.5248x over previous
import jax
import jax.numpy as jnp
from jax import lax
from jax.experimental import pallas as pl
from jax.experimental.pallas import tpu as pltpu
from jax.experimental.pallas import tpu_sc as plsc

_B, _C, _H, _W = 128, 3, 224, 224
_PX = _H * _W
_Q = 56
_NBUF = 2

_info = plsc.get_sparse_core_info()
_NC, _NS = _info.num_cores, _info.num_subcores
_NW = _NC * _NS
_PPW = _PX // _NW
_TOT = _PPW // _Q
_GROUPS = _TOT // _NBUF


def _sc_body(x_hbm, w_hbm, o_hbm, wv, ibuf, obuf, isem, osem):
    wid = lax.axis_index("s") * _NC + lax.axis_index("c")
    base_px = wid * _PPW
    pltpu.sync_copy(w_hbm, wv)
    msels = [wv[pl.ds(l * 16, 16)] > 0.5 for l in range(8)]

    for b in range(_NBUF):
        p0 = base_px + b * _Q
        pltpu.make_async_copy(
            x_hbm.at[:, pl.ds(p0, _Q), :], ibuf.at[b], isem.at[b]).start()

    def group(g, carry):
        for b in range(_NBUF):
            c = g * _NBUF + b
            p0 = base_px + c * _Q
            pltpu.make_async_copy(
                x_hbm.at[:, pl.ds(p0, _Q), :], ibuf.at[b], isem.at[b]).wait()

            @pl.when(c >= _NBUF)
            def _(b=b, c=c):
                pp0 = base_px + (c - _NBUF) * _Q
                pltpu.make_async_copy(
                    obuf.at[b], o_hbm.at[:, pl.ds(pp0, _Q), :],
                    osem.at[b]).wait()

            def px(p, pcarry, b=b):
                for l in range(8):
                    l0 = l * 16
                    r = ibuf[b, 0, p, pl.ds(l0, 16)]
                    g_ = ibuf[b, 1, p, pl.ds(l0, 16)]
                    b_ = ibuf[b, 2, p, pl.ds(l0, 16)]
                    lum = (r * (299.0 / 1000.0) + g_ * (587.0 / 1000.0)
                           + b_ * (114.0 / 1000.0))
                    obuf[b, 0, p, pl.ds(l0, 16)] = jnp.where(msels[l], lum, r)
                    obuf[b, 1, p, pl.ds(l0, 16)] = jnp.where(msels[l], lum, g_)
                    obuf[b, 2, p, pl.ds(l0, 16)] = jnp.where(msels[l], lum, b_)
                return pcarry

            lax.fori_loop(0, _Q, px, 0)

            pltpu.make_async_copy(
                obuf.at[b], o_hbm.at[:, pl.ds(p0, _Q), :], osem.at[b]).start()

            @pl.when(c + _NBUF < _TOT)
            def _(b=b, c=c):
                np0 = base_px + (c + _NBUF) * _Q
                pltpu.make_async_copy(
                    x_hbm.at[:, pl.ds(np0, _Q), :], ibuf.at[b],
                    isem.at[b]).start()

        return carry

    lax.fori_loop(0, _GROUPS, group, 0)

    for b in range(_NBUF):
        p0 = base_px + (_TOT - _NBUF + b) * _Q
        pltpu.make_async_copy(
            obuf.at[b], o_hbm.at[:, pl.ds(p0, _Q), :], osem.at[b]).wait()


@jax.jit
def _sc_gray(xt, w):
    mesh = plsc.VectorSubcoreMesh(core_axis_name="c", subcore_axis_name="s")
    return pl.kernel(
        _sc_body,
        out_type=jax.ShapeDtypeStruct((_C, _PX, _B), jnp.float32),
        mesh=mesh,
        scratch_types=[
            pltpu.VMEM((_B,), jnp.float32),
            pltpu.VMEM((_NBUF, _C, _Q, _B), jnp.float32),
            pltpu.VMEM((_NBUF, _C, _Q, _B), jnp.float32),
            pltpu.SemaphoreType.DMA((_NBUF,)),
            pltpu.SemaphoreType.DMA((_NBUF,)),
        ],
    )(xt, w)


def kernel(x, inds):
    xt = jnp.transpose(x, (1, 2, 3, 0)).reshape(_C, _PX, _B)
    out_t = _sc_gray(xt, inds.astype(jnp.float32))
    return jnp.transpose(out_t.reshape(_C, _H, _W, _B), (3, 0, 1, 2))


# speedup vs baseline: 5.9231x; 1.4685x over previous
import jax
import jax.numpy as jnp
from jax.experimental import pallas as pl
from jax.experimental.pallas import tpu as pltpu

_B, _C, _H, _W = 128, 3, 224, 224
_PX = _H * _W
_PB = 3136
_G = _PX // _PB


def _tc_body(m_ref, x_ref, o_ref):
    mb = m_ref[...] > 0.5
    r = x_ref[0]
    g = x_ref[1]
    b = x_ref[2]
    lum = (r * (299.0 / 1000.0) + g * (587.0 / 1000.0)
           + b * (114.0 / 1000.0))
    o_ref[0] = jnp.where(mb, lum, r)
    o_ref[1] = jnp.where(mb, lum, g)
    o_ref[2] = jnp.where(mb, lum, b)


def kernel(x, inds):
    xt = jnp.transpose(x, (1, 2, 3, 0)).reshape(_C, _PX, _B)
    m = inds.astype(jnp.float32).reshape(1, _B)
    out_t = pl.pallas_call(
        _tc_body,
        grid=(_G,),
        in_specs=[
            pl.BlockSpec((1, _B), lambda i: (0, 0)),
            pl.BlockSpec((_C, _PB, _B), lambda i: (0, i, 0)),
        ],
        out_specs=pl.BlockSpec((_C, _PB, _B), lambda i: (0, i, 0)),
        out_shape=jax.ShapeDtypeStruct((_C, _PX, _B), jnp.float32),
    )(m, xt)
    return jnp.transpose(out_t.reshape(_C, _H, _W, _B), (3, 0, 1, 2))
